# Initial kernel scaffold; baseline (speedup 1.0000x reference)
import jax
import jax.numpy as jnp
from jax.experimental import pallas as pl


def kernel(z, codebook):
    raise NotImplementedError("write your pallas kernel here")


# speedup vs baseline: 1.1036x; 1.1036x over previous
import functools

import jax
import jax.numpy as jnp
from jax import lax
from jax.experimental import pallas as pl
from jax.experimental.pallas import tpu as pltpu
from jax.experimental.pallas import tpu_sc as plsc

NUM_E = 8192
DIM = 256
N_TOK = 16384
BM = 1024
BN = 1024
W0, W1 = 2736, 5472
_BIG = 2**30


def _bf16_rne(x):
    b = lax.bitcast_convert_type(x, jnp.uint32)
    r = (b + jnp.uint32(0x7FFF) + ((b >> 16) & jnp.uint32(1))) & jnp.uint32(0xFFFF0000)
    return lax.bitcast_convert_type(r, jnp.float32)


def _chunk_minargmin(d, base, mask=None):
    iot = lax.broadcasted_iota(jnp.int32, d.shape, 1) + base
    if mask is not None:
        d = jnp.where(mask, d, jnp.inf)
    cmin = jnp.min(d, axis=1)
    cidx = jnp.min(jnp.where(d == cmin[:, None], iot, _BIG), axis=1)
    return cmin, cidx


def _merge(bv, bi, cv, ci):
    take = cv < bv
    return jnp.where(take, cv, bv), jnp.where(take, ci, bi)


def _dist_argmin_body(r1_ref, r2_ref, z_ref, cb_ref, idx_ref, dpick_ref):
    zb = z_ref[...]
    r1b = r1_ref[...]

    inf1 = jnp.full((BM,), jnp.inf, jnp.float32)
    zero1 = jnp.zeros((BM,), jnp.int32)
    wins = [(inf1, zero1), (inf1, zero1), (inf1, zero1)]

    for c in range(NUM_E // BN):
        base = c * BN
        cchunk = cb_ref[pl.ds(base, BN), :]
        r2c = r2_ref[:, pl.ds(base, BN)]
        m = lax.dot_general(zb, cchunk, (((1,), (1,)), ((), ())),
                            preferred_element_type=jnp.float32)
        d = (r1b + r2c) - 2.0 * m
        for w, (wlo, whi) in enumerate(((0, W0), (W0, W1), (W1, NUM_E))):
            lo, hi = max(base, wlo), min(base + BN, whi)
            if lo >= hi:
                continue
            if lo == base and hi == base + BN:
                cv, ci = _chunk_minargmin(d, base)
            else:
                col = lax.broadcasted_iota(jnp.int32, d.shape, 1) + base
                msk = (col >= lo) & (col < hi)
                cv, ci = _chunk_minargmin(d, base, msk)
            wins[w] = _merge(wins[w][0], wins[w][1], cv, ci)

    (v0, i0), (v1, i1), (v2, i2) = wins
    accv = _bf16_rne(v0)
    acci = i0
    pickv = v0
    take = v1 < accv
    accv = jnp.where(take, _bf16_rne(v1), accv)
    acci = jnp.where(take, i1, acci)
    pickv = jnp.where(take, v1, pickv)
    take = v2 < accv
    acci = jnp.where(take, i2, acci)
    pickv = jnp.where(take, v2, pickv)

    idx_ref[...] = acci
    dpick_ref[...] = pickv


def _dist_argmin(r1, r2, z_flat, codebook):
    return pl.pallas_call(
        _dist_argmin_body,
        grid=(N_TOK // BM,),
        in_specs=[
            pl.BlockSpec((BM, 1), lambda i: (i, 0)),
            pl.BlockSpec((1, NUM_E), lambda i: (0, 0)),
            pl.BlockSpec((BM, DIM), lambda i: (i, 0)),
            pl.BlockSpec((NUM_E, DIM), lambda i: (0, 0)),
        ],
        out_specs=[
            pl.BlockSpec((BM,), lambda i: (i,)),
            pl.BlockSpec((BM,), lambda i: (i,)),
        ],
        out_shape=[
            jax.ShapeDtypeStruct((N_TOK,), jnp.int32),
            jax.ShapeDtypeStruct((N_TOK,), jnp.float32),
        ],
    )(r1, r2, z_flat, codebook)


_SC_CHUNK = 128


def _sc_gather(codebook, idx):
    info = plsc.get_sparse_core_info()
    nw = info.num_cores * info.num_subcores
    b_per_w = N_TOK // nw
    nchunk = b_per_w // _SC_CHUNK
    mesh = plsc.VectorSubcoreMesh(core_axis_name="c", subcore_axis_name="s")

    @functools.partial(
        pl.kernel, mesh=mesh,
        out_type=jax.ShapeDtypeStruct((N_TOK, DIM), jnp.float32),
        scratch_types=[
            pltpu.VMEM((_SC_CHUNK,), jnp.int32),
            pltpu.VMEM((_SC_CHUNK, DIM), jnp.float32),
            pltpu.SemaphoreType.DMA,
        ],
    )
    def gather(table_hbm, idx_hbm, out_hbm, idx_v, rows_v, sem):
        wid = lax.axis_index("s") * info.num_cores + lax.axis_index("c")
        base = wid * b_per_w
        for t in range(nchunk):
            off = base + t * _SC_CHUNK
            pltpu.sync_copy(idx_hbm.at[pl.ds(off, _SC_CHUNK)], idx_v)
            pltpu.async_copy(table_hbm.at[idx_v], rows_v, sem).wait()
            pltpu.sync_copy(rows_v, out_hbm.at[pl.ds(off, _SC_CHUNK)])

    return gather(codebook, idx)


def kernel(z, codebook):
    zz = z[0]
    z_flat = zz.reshape(-1, DIM)
    r1 = jnp.sum(zz ** 2, axis=-1).reshape(-1, 1)
    r2 = jnp.sum(codebook ** 2, axis=1).reshape(1, -1)
    idx, dpick = _dist_argmin(r1, r2, z_flat, codebook)
    zq = _sc_gather(codebook, idx)
    z_out = zq.reshape(zz.shape)
    vq_loss = 1.1 * (jnp.sum(dpick) / jnp.float32(z_flat.size))
    return (z_out, vq_loss)
